# baseline (speedup 1.0000x reference)
import functools

import jax
import jax.numpy as jnp
from jax import lax
from jax.experimental import pallas as pl
from jax.experimental.pallas import tpu as pltpu
from jax.experimental.pallas import tpu_sc as plsc

SEQ = 200
DIM = 64
PAD_DIM = 128
NUM_CORES = 2
NUM_SUBCORES = 16
NUM_WORKERS = NUM_CORES * NUM_SUBCORES
BW = 128
NSLOT = 2


PITCH = 129


def _body(table_hbm, idxt_hbm, pos_hbm, out_hbm, idx_v, gbuf, sbuf, pos_v,
          tbuf, gsems, ssems):
    wid = lax.axis_index("s") * NUM_CORES + lax.axis_index("c")
    b0 = wid * BW
    lanes = lax.iota(jnp.int32, 16)
    lanep = lanes * PITCH

    pltpu.sync_copy(pos_hbm, pos_v)
    pltpu.sync_copy(idxt_hbm.at[:, pl.ds(b0, BW)], idx_v)

    def fire_gather(s, slot):
        pltpu.async_copy(
            table_hbm.at[idx_v.at[s]], gbuf.at[slot], gsems.at[slot],
        )

    def wait_gather(slot):
        pltpu.make_async_copy(
            table_hbm.at[idx_v.at[0]], gbuf.at[slot], gsems.at[slot],
        ).wait()

    def fire_store(s, slot):
        pltpu.async_copy(
            sbuf.at[slot], out_hbm.at[s, :, pl.ds(b0, BW)], ssems.at[slot],
        )

    def wait_store(slot):
        pltpu.make_async_copy(
            sbuf.at[slot], out_hbm.at[0, :, pl.ds(b0, BW)], ssems.at[slot],
        ).wait()

    for slot in range(NSLOT):
        fire_gather(slot, slot)

    @pl.loop(0, SEQ // NSLOT)
    def _pos_loop(i):
        for slot in range(NSLOT):
            s = i * NSLOT + slot
            wait_gather(slot)

            @pl.when(i >= 1)
            def _():
                wait_store(slot)

            pos_rows = [pos_v[s, pl.ds(c * 16, 16)] for c in range(DIM // 16)]

            @plsc.parallel_loop(0, BW, unroll=8)
            def _b_loop(b):
                for c in range(DIM // 16):
                    tok = gbuf[slot, b, pl.ds(c * 16, 16)]
                    addr = lanep + (b + c * 16 * PITCH)
                    plsc.store_scatter(tbuf, [addr], tok + pos_rows[c])

            @plsc.parallel_loop(0, DIM, unroll=8)
            def _d_loop(d):
                base = d * PITCH
                for k in range(BW // 16):
                    v = plsc.load_gather(tbuf, [lanes + (base + k * 16)])
                    sbuf[slot, d, pl.ds(k * 16, 16)] = v

            @pl.when(i < SEQ // NSLOT - 1)
            def _():
                fire_gather(s + NSLOT, slot)

            fire_store(s, slot)

    for slot in range(NSLOT):
        wait_store(slot)


def kernel(inputs, token_table, pos_table):
    batch, seq = inputs.shape
    idx_t = inputs.T
    table_pad = jnp.pad(token_table, ((0, 0), (0, PAD_DIM - DIM)))
    pos_pad = jnp.pad(pos_table, ((0, 0), (0, PAD_DIM - DIM)))
    mesh = plsc.VectorSubcoreMesh(
        core_axis_name="c",
        subcore_axis_name="s",
        num_cores=NUM_CORES,
        num_subcores=NUM_SUBCORES,
    )
    out_t = pl.kernel(
        _body,
        out_type=jax.ShapeDtypeStruct((seq, DIM, batch), jnp.float32),
        mesh=mesh,
        scratch_types=[
            pltpu.VMEM((SEQ, BW), jnp.int32),
            pltpu.VMEM((NSLOT, BW, PAD_DIM), jnp.float32),
            pltpu.VMEM((NSLOT, DIM, BW), jnp.float32),
            pltpu.VMEM((SEQ, PAD_DIM), jnp.float32),
            pltpu.VMEM((DIM * PITCH + 16,), jnp.float32),
            pltpu.SemaphoreType.DMA((NSLOT,)),
            pltpu.SemaphoreType.DMA((NSLOT,)),
        ],
        compiler_params=pltpu.CompilerParams(
            use_tc_tiling_on_sc=True, needs_layout_passes=False,
        ),
    )(table_pad, idx_t, pos_pad)
    return jnp.transpose(out_t, (2, 0, 1))


# speedup vs baseline: 1.0017x; 1.0017x over previous
import jax
import jax.numpy as jnp
from jax import lax
from jax.experimental import pallas as pl
from jax.experimental.pallas import tpu as pltpu
from jax.experimental.pallas import tpu_sc as plsc

SEQ = 200
DIM = 64
PAD_DIM = 128
NUM_CORES = 2
NUM_SUBCORES = 16
NUM_WORKERS = NUM_CORES * NUM_SUBCORES
BW = 128
NSLOT = 2


PITCH = 129


def _body(table_hbm, idxt_hbm, pos_hbm, out_hbm, idx_v, gbuf, sbuf, pos_v,
          tbuf, gsems, ssems):
    wid = lax.axis_index("s") * NUM_CORES + lax.axis_index("c")
    b0 = wid * BW
    lanes = lax.iota(jnp.int32, 16)
    lanep = lanes * PITCH

    pltpu.sync_copy(pos_hbm, pos_v)
    pltpu.sync_copy(idxt_hbm.at[:, pl.ds(b0, BW)], idx_v)

    def fire_gather(s, slot):
        pltpu.async_copy(
            table_hbm.at[idx_v.at[s]], gbuf.at[slot], gsems.at[slot],
        )

    def wait_gather(slot):
        pltpu.make_async_copy(
            table_hbm.at[idx_v.at[0]], gbuf.at[slot], gsems.at[slot],
        ).wait()

    def fire_store(s, slot):
        pltpu.async_copy(
            sbuf.at[slot], out_hbm.at[s, :, pl.ds(b0, BW)], ssems.at[slot],
        )

    def wait_store(slot):
        pltpu.make_async_copy(
            sbuf.at[slot], out_hbm.at[0, :, pl.ds(b0, BW)], ssems.at[slot],
        ).wait()

    for slot in range(NSLOT):
        fire_gather(slot, slot)

    @pl.loop(0, SEQ // NSLOT)
    def _pos_loop(i):
        for slot in range(NSLOT):
            s = i * NSLOT + slot
            wait_gather(slot)

            @pl.when(i >= 1)
            def _():
                wait_store(slot)

            pos_rows = [pos_v[s, pl.ds(c * 16, 16)] for c in range(DIM // 16)]

            @plsc.parallel_loop(0, BW, unroll=8)
            def _b_loop(b):
                for c in range(DIM // 16):
                    tok = gbuf[slot, b, pl.ds(c * 16, 16)]
                    addr = lanep + (b + c * 16 * PITCH)
                    plsc.store_scatter(tbuf, [addr], tok + pos_rows[c])

            @plsc.parallel_loop(0, DIM, unroll=8)
            def _d_loop(d):
                base = d * PITCH
                for k in range(BW // 16):
                    v = plsc.load_gather(tbuf, [lanes + (base + k * 16)])
                    sbuf[slot, d, pl.ds(k * 16, 16)] = v

            @pl.when(i < SEQ // NSLOT - 1)
            def _():
                fire_gather(s + NSLOT, slot)

            fire_store(s, slot)

    for slot in range(NSLOT):
        wait_store(slot)


def kernel(inputs, token_table, pos_table):
    batch, seq = inputs.shape
    idx_t = inputs.T
    table_pad = jnp.pad(token_table, ((0, 0), (0, PAD_DIM - DIM)))
    pos_pad = jnp.pad(pos_table, ((0, 0), (0, PAD_DIM - DIM)))
    mesh = plsc.VectorSubcoreMesh(
        core_axis_name="c",
        subcore_axis_name="s",
        num_cores=NUM_CORES,
        num_subcores=NUM_SUBCORES,
    )
    out_t = pl.kernel(
        _body,
        out_type=jax.ShapeDtypeStruct((seq, DIM, batch), jnp.float32),
        mesh=mesh,
        scratch_types=[
            pltpu.VMEM((SEQ, BW), jnp.int32),
            pltpu.VMEM((NSLOT, BW, PAD_DIM), jnp.float32),
            pltpu.VMEM((NSLOT, DIM, BW), jnp.float32),
            pltpu.VMEM((SEQ, PAD_DIM), jnp.float32),
            pltpu.VMEM((DIM * PITCH + 16,), jnp.float32),
            pltpu.SemaphoreType.DMA((NSLOT,)),
            pltpu.SemaphoreType.DMA((NSLOT,)),
        ],
        compiler_params=pltpu.CompilerParams(
            use_tc_tiling_on_sc=True, needs_layout_passes=False,
        ),
    )(table_pad, idx_t, pos_pad)
    return jnp.transpose(out_t, (2, 0, 1))
